# Initial kernel scaffold; baseline (speedup 1.0000x reference)
import jax
import jax.numpy as jnp
from jax.experimental import pallas as pl


def kernel(x, pos, edge_index, edge_attr, batch, x1, frag_levels, adduct_feats, params):
    raise NotImplementedError("write your pallas kernel here")


# speedup vs baseline: 1.0004x; 1.0004x over previous
import jax
import jax.numpy as jnp
from jax.experimental import pallas as pl
from jax.experimental.pallas import tpu as pltpu

N_NODES = 10000
B = 64
HID = 128
OUT_NODE = 1024
HEAD_IN = 1152
PROP = 1000


def _silu(x):
    return x * jax.nn.sigmoid(x)


def _head_body(z_ref, w1_ref, b1_ref, w2_ref, b2_ref, wo_ref, bo_ref, o_ref):
    z = z_ref[...]
    r = _silu(z @ w1_ref[...] + b1_ref[...])
    z2 = r @ w2_ref[...] + b2_ref[...] + z
    o_ref[...] = z2 @ wo_ref[...] + bo_ref[...]


def _head(z, params):
    return pl.pallas_call(
        _head_body,
        out_shape=jax.ShapeDtypeStruct((B, PROP), jnp.float32),
    )(z, params["res1"]["W"], params["res1"]["b"][None, :],
      params["res2"]["W"], params["res2"]["b"][None, :],
      params["out"]["W"], params["out"]["b"][None, :])


def kernel(x, pos, edge_index, edge_attr, batch, x1, frag_levels, adduct_feats, params):
    row, col = edge_index[0], edge_index[1]
    h = x @ params["emb_in"]["W"] + params["emb_in"]["b"]
    coord = pos
    for layer in params["layers"]:
        coord_diff = coord[row] - coord[col]
        radial = jnp.sum(coord_diff ** 2, axis=1, keepdims=True)
        e_in = jnp.concatenate([h[row], h[col], radial, edge_attr], axis=1)
        m = _silu(e_in @ layer["edge1"]["W"] + layer["edge1"]["b"])
        m = _silu(m @ layer["edge2"]["W"] + layer["edge2"]["b"])
        w = _silu(m @ layer["coord1"]["W"] + layer["coord1"]["b"]) @ layer["coord2"]["W"]
        trans = coord_diff * w
        agg = jax.ops.segment_sum(trans, row, num_segments=N_NODES)
        cnt = jax.ops.segment_sum(jnp.ones((row.shape[0], 1), jnp.float32), row,
                                  num_segments=N_NODES)
        coord = coord + agg / jnp.maximum(cnt, 1.0)
        m_agg = jax.ops.segment_sum(m, row, num_segments=N_NODES)
        n_in = jnp.concatenate([h, m_agg], axis=1)
        h = h + _silu(n_in @ layer["node1"]["W"] + layer["node1"]["b"]) @ layer["node2"]["W"] + layer["node2"]["b"]
    h = h @ params["emb_out"]["W"] + params["emb_out"]["b"]
    pooled = jax.ops.segment_max(h, batch, num_segments=B)
    z = jnp.concatenate([pooled, x1, frag_levels.reshape(B, 32),
                         adduct_feats.reshape(B, 32)], axis=1)
    return _head(z, params)


# speedup vs baseline: 2.0300x; 2.0291x over previous
import functools

import jax
import jax.numpy as jnp
from jax import lax
from jax.experimental import pallas as pl
from jax.experimental.pallas import tpu as pltpu
from jax.experimental.pallas import tpu_sc as plsc

N = 10000
NP = 10112
E = 320000
EP = 323584
HID = 128
BGRP = 64
ONODE = 1024
PROP = 1000

NW = 32
SCH = EP // (NW * 128)
ZROWS = NP // 16

BE = 512
NEB = EP // BE
BN = 400
NNB = N // BN

_mesh = plsc.VectorSubcoreMesh(core_axis_name="c", subcore_axis_name="s")
_sc_params = pltpu.CompilerParams(needs_layout_passes=False)


def _silu(x):
    return x * jax.nn.sigmoid(x)


@functools.partial(
    pl.kernel,
    out_type=[
        jax.ShapeDtypeStruct((EP, HID), jnp.float32),
        jax.ShapeDtypeStruct((EP, HID), jnp.float32),
        jax.ShapeDtypeStruct((EP * 8,), jnp.float32),
    ],
    mesh=_mesh,
    compiler_params=_sc_params,
    scratch_types=[
        pltpu.VMEM((SCH, 128), jnp.int32),
        pltpu.VMEM((SCH, 128), jnp.int32),
        pltpu.VMEM((128, HID), jnp.float32),
        pltpu.VMEM((1024,), jnp.float32),
        pltpu.VMEM((NP * 8,), jnp.float32),
        pltpu.SemaphoreType.DMA,
    ],
)
def _sc_gather(tab_hbm, cflat_hbm, rowi_hbm, colni_hbm,
               gfa_hbm, gfb_hbm, aux_hbm,
               rowi_v, colni_v, feat_v, aux_v, cflat_v, sem):
    c = lax.axis_index("c")
    s = lax.axis_index("s")
    wid = s * 2 + c
    pltpu.sync_copy(cflat_hbm, cflat_v)
    pltpu.sync_copy(rowi_hbm.at[wid], rowi_v)
    pltpu.sync_copy(colni_hbm.at[wid], colni_v)
    zero16 = jnp.zeros((16,), jnp.float32)
    for i in range(64):
        aux_v[pl.ds(i * 16, 16)] = zero16
    iota = lax.iota(jnp.int32, 16)

    def body(k, _):
        base = wid * SCH * 128 + k * 128
        pltpu.async_copy(tab_hbm.at[rowi_v.at[k]], feat_v, sem).wait()
        pltpu.sync_copy(feat_v, gfa_hbm.at[pl.ds(base, 128)])
        pltpu.async_copy(tab_hbm.at[colni_v.at[k]], feat_v, sem).wait()
        pltpu.sync_copy(feat_v, gfb_hbm.at[pl.ds(base, 128)])
        for g in range(8):
            r16 = rowi_v[k, pl.ds(g * 16, 16)] * 8
            c16 = colni_v[k, pl.ds(g * 16, 16)] * 8 - (8 * N)
            dx = plsc.load_gather(cflat_v, [r16]) - plsc.load_gather(cflat_v, [c16])
            dy = (plsc.load_gather(cflat_v, [r16 + 1])
                  - plsc.load_gather(cflat_v, [c16 + 1]))
            dz = (plsc.load_gather(cflat_v, [r16 + 2])
                  - plsc.load_gather(cflat_v, [c16 + 2]))
            rad = dx * dx + dy * dy + dz * dz
            j8 = (g * 16 + iota) * 8
            plsc.store_scatter(aux_v, [j8], dx)
            plsc.store_scatter(aux_v, [j8 + 1], dy)
            plsc.store_scatter(aux_v, [j8 + 2], dz)
            plsc.store_scatter(aux_v, [j8 + 3], rad)
        pltpu.sync_copy(aux_v, aux_hbm.at[pl.ds(base * 8, 1024)])
        return 0

    lax.fori_loop(0, SCH, body, 0)


@functools.partial(
    pl.kernel,
    out_type=jax.ShapeDtypeStruct((2, NP, HID), jnp.float32),
    mesh=_mesh,
    compiler_params=_sc_params,
    scratch_types=[
        pltpu.VMEM((SCH, 128), jnp.int32),
        pltpu.VMEM((128, HID), jnp.float32),
        pltpu.VMEM_SHARED((NP, HID), jnp.float32),
        pltpu.SemaphoreType.DMA,
    ],
)
def _sc_scatter_m(msg_hbm, rowi_hbm, zf_hbm, mout_hbm, rowi_v, upd_v, accm, sem):
    c = lax.axis_index("c")
    s = lax.axis_index("s")
    wid = s * 2 + c
    pltpu.sync_copy(zf_hbm.at[pl.ds(s * ZROWS, ZROWS)],
                    accm.at[pl.ds(s * ZROWS, ZROWS)])
    pltpu.sync_copy(rowi_hbm.at[wid], rowi_v)
    plsc.subcore_barrier()

    def body(k, _):
        base = wid * SCH * 128 + k * 128
        pltpu.sync_copy(msg_hbm.at[pl.ds(base, 128)], upd_v)
        pltpu.sync_copy(upd_v, accm.at[rowi_v.at[k]], add=True)
        return 0

    lax.fori_loop(0, SCH, body, 0)
    plsc.subcore_barrier()
    pltpu.sync_copy(accm.at[pl.ds(s * ZROWS, ZROWS)],
                    mout_hbm.at[c, pl.ds(s * ZROWS, ZROWS)])


@functools.partial(
    pl.kernel,
    out_type=jax.ShapeDtypeStruct((NW, NP * 8), jnp.float32),
    mesh=_mesh,
    compiler_params=_sc_params,
    scratch_types=[
        pltpu.VMEM((SCH, 128), jnp.int32),
        pltpu.VMEM((1024,), jnp.float32),
        pltpu.VMEM((NP * 8,), jnp.float32),
        pltpu.SemaphoreType.DMA,
    ],
)
def _sc_scatter_aux(aux_hbm, rowi_hbm, za_hbm, aout_hbm,
                    rowi_v, aux_v, acca_v, sem):
    c = lax.axis_index("c")
    s = lax.axis_index("s")
    wid = s * 2 + c
    pltpu.sync_copy(za_hbm, acca_v)
    pltpu.sync_copy(rowi_hbm.at[wid], rowi_v)
    iota = lax.iota(jnp.int32, 16)

    def body(k, _):
        base = wid * SCH * 128 + k * 128
        pltpu.sync_copy(aux_hbm.at[pl.ds(base * 8, 1024)], aux_v)
        for g in range(8):
            n8 = rowi_v[k, pl.ds(g * 16, 16)] * 8
            j8 = (g * 16 + iota) * 8
            for comp in range(4):
                vals = plsc.load_gather(aux_v, [j8 + comp])
                plsc.addupdate_scatter(acca_v, [n8 + comp], vals)
        return 0

    lax.fori_loop(0, SCH, body, 0)
    pltpu.sync_copy(acca_v, aout_hbm.at[wid])


def _init_body(x_ref, pos_ref, wemb_ref, bemb_ref, w1a_ref, w1b_ref, b1_ref,
               h_ref, c_ref, t_ref):
    h = x_ref[...] @ wemb_ref[...] + bemb_ref[...]
    h_ref[...] = h
    c_ref[...] = pos_ref[...]
    t_ref[0] = h @ w1a_ref[...] + b1_ref[...]
    t_ref[1] = h @ w1b_ref[...]


def _tc_init(x, pospad, p):
    return pl.pallas_call(
        _init_body,
        grid=(NNB,),
        in_specs=[
            pl.BlockSpec((BN, HID), lambda i: (i, 0)),
            pl.BlockSpec((BN, 8), lambda i: (i, 0)),
            pl.BlockSpec((HID, HID), lambda i: (0, 0)),
            pl.BlockSpec((1, HID), lambda i: (0, 0)),
            pl.BlockSpec((HID, HID), lambda i: (0, 0)),
            pl.BlockSpec((HID, HID), lambda i: (0, 0)),
            pl.BlockSpec((1, HID), lambda i: (0, 0)),
        ],
        out_specs=[
            pl.BlockSpec((BN, HID), lambda i: (i, 0)),
            pl.BlockSpec((BN, 8), lambda i: (i, 0)),
            pl.BlockSpec((2, BN, HID), lambda i: (0, i, 0)),
        ],
        out_shape=[
            jax.ShapeDtypeStruct((N, HID), jnp.float32),
            jax.ShapeDtypeStruct((NP, 8), jnp.float32),
            jax.ShapeDtypeStruct((2, N, HID), jnp.float32),
        ],
    )(x, pospad, p["emb_in"]["W"], p["emb_in"]["b"][None, :],
      p["w1a"], p["w1b"], p["b1"])


def _edge_body(ga_ref, gb_ref, aux_ref, ea_ref, r8_ref, we_ref,
               w2_ref, b2_ref, wc1_ref, bc1_ref, wc2_ref, m_out, t_out):
    aux = aux_ref[...]
    pre = (ga_ref[...] + gb_ref[...] + aux @ r8_ref[...]
           + ea_ref[...] @ we_ref[...])
    t1 = _silu(pre)
    m = _silu(t1 @ w2_ref[...] + b2_ref[...])
    c1 = _silu(m @ wc1_ref[...] + bc1_ref[...])
    w8 = c1 @ wc2_ref[...]
    t8 = aux * w8
    lane = lax.broadcasted_iota(jnp.int32, (BE, 8), 1)
    t8 = jnp.where(lane == 3, 1.0, t8)
    m_out[...] = m
    t_out[...] = t8


def _tc_edge(gfa, gfb, aux1, ea_pad, lw):
    cst = lambda i: (0, 0)
    return pl.pallas_call(
        _edge_body,
        grid=(NEB,),
        in_specs=[
            pl.BlockSpec((BE, HID), lambda i: (i, 0)),
            pl.BlockSpec((BE, HID), lambda i: (i, 0)),
            pl.BlockSpec((BE, 8), lambda i: (i, 0)),
            pl.BlockSpec((BE, 8), lambda i: (i, 0)),
            pl.BlockSpec((8, HID), cst),
            pl.BlockSpec((8, HID), cst),
            pl.BlockSpec((HID, HID), cst),
            pl.BlockSpec((1, HID), cst),
            pl.BlockSpec((HID, HID), cst),
            pl.BlockSpec((1, HID), cst),
            pl.BlockSpec((HID, 8), cst),
        ],
        out_specs=[
            pl.BlockSpec((BE, HID), lambda i: (i, 0)),
            pl.BlockSpec((BE, 8), lambda i: (i, 0)),
        ],
        out_shape=[
            jax.ShapeDtypeStruct((EP, HID), jnp.float32),
            jax.ShapeDtypeStruct((EP, 8), jnp.float32),
        ],
    )(gfa, gfb, aux1, ea_pad, lw["r8"], lw["we8"], lw["w2"],
      lw["b2"], lw["wc1"], lw["bc1"], lw["wc2_8"])


def _node_body(h_ref, c_ref, pm_ref, pa_ref, wn1a_ref, wn1b_ref, bn1_ref,
               wn2_ref, bn2_ref, w1a_ref, w1b_ref, b1_ref,
               h_out, c_out, t_out):
    h = h_ref[...]
    magg = pm_ref[0] + pm_ref[1]
    t = jnp.sum(pa_ref[...], axis=0)
    cnt = jnp.maximum(t[:, 3:4], 1.0)
    lane = lax.broadcasted_iota(jnp.int32, (BN, 8), 1)
    c_out[...] = c_ref[...] + jnp.where(lane < 3, t, 0.0) / cnt
    n1 = _silu(h @ wn1a_ref[...] + magg @ wn1b_ref[...] + bn1_ref[...])
    hnew = h + n1 @ wn2_ref[...] + bn2_ref[...]
    h_out[...] = hnew
    t_out[0] = hnew @ w1a_ref[...] + b1_ref[...]
    t_out[1] = hnew @ w1b_ref[...]


def _tc_node(h, coord, pm, pa, lw, nxt):
    cst = lambda i: (0, 0)
    return pl.pallas_call(
        _node_body,
        grid=(NNB,),
        in_specs=[
            pl.BlockSpec((BN, HID), lambda i: (i, 0)),
            pl.BlockSpec((BN, 8), lambda i: (i, 0)),
            pl.BlockSpec((2, BN, HID), lambda i: (0, i, 0)),
            pl.BlockSpec((NW, BN, 8), lambda i: (0, i, 0)),
            pl.BlockSpec((HID, HID), cst),
            pl.BlockSpec((HID, HID), cst),
            pl.BlockSpec((1, HID), cst),
            pl.BlockSpec((HID, HID), cst),
            pl.BlockSpec((1, HID), cst),
            pl.BlockSpec((HID, HID), cst),
            pl.BlockSpec((HID, HID), cst),
            pl.BlockSpec((1, HID), cst),
        ],
        out_specs=[
            pl.BlockSpec((BN, HID), lambda i: (i, 0)),
            pl.BlockSpec((BN, 8), lambda i: (i, 0)),
            pl.BlockSpec((2, BN, HID), lambda i: (0, i, 0)),
        ],
        out_shape=[
            jax.ShapeDtypeStruct((N, HID), jnp.float32),
            jax.ShapeDtypeStruct((NP, 8), jnp.float32),
            jax.ShapeDtypeStruct((2, N, HID), jnp.float32),
        ],
    )(h, coord, pm, pa, lw["wn1a"], lw["wn1b"], lw["bn1"], lw["wn2"],
      lw["bn2"], nxt["w1a"], nxt["w1b"], nxt["b1"])


def _pool_body(h_ref, pm_ref, bat_ref, wn1a_ref, wn1b_ref, bn1_ref, wn2_ref,
               bn2_ref, wo_ref, bo_ref, out_ref):
    @pl.when(pl.program_id(0) == 0)
    def _():
        out_ref[...] = jnp.full((BGRP, ONODE), -jnp.inf, jnp.float32)

    h = h_ref[...]
    magg = pm_ref[0] + pm_ref[1]
    n1 = _silu(h @ wn1a_ref[...] + magg @ wn1b_ref[...] + bn1_ref[...])
    hf = h + n1 @ wn2_ref[...] + bn2_ref[...]
    z = hf @ wo_ref[...] + bo_ref[...]
    bb = bat_ref[:, :1]
    blo = bat_ref[0, 0]
    bhi = bat_ref[BN - 1, 0]

    def body(b, _):
        zz = jnp.where(bb == b, z, -jnp.inf)
        mv = jnp.max(zz, axis=0, keepdims=True)
        out_ref[pl.ds(b, 1), :] = jnp.maximum(out_ref[pl.ds(b, 1), :], mv)
        return 0

    lax.fori_loop(blo, bhi + 1, body, 0)


def _tc_pool(h, pm, bat8, lw, p):
    cst = lambda i: (0, 0)
    return pl.pallas_call(
        _pool_body,
        grid=(NNB,),
        in_specs=[
            pl.BlockSpec((BN, HID), lambda i: (i, 0)),
            pl.BlockSpec((2, BN, HID), lambda i: (0, i, 0)),
            pl.BlockSpec((BN, 8), lambda i: (i, 0)),
            pl.BlockSpec((HID, HID), cst),
            pl.BlockSpec((HID, HID), cst),
            pl.BlockSpec((1, HID), cst),
            pl.BlockSpec((HID, HID), cst),
            pl.BlockSpec((1, HID), cst),
            pl.BlockSpec((HID, ONODE), cst),
            pl.BlockSpec((1, ONODE), cst),
        ],
        out_specs=pl.BlockSpec((BGRP, ONODE), lambda i: (0, 0)),
        out_shape=jax.ShapeDtypeStruct((BGRP, ONODE), jnp.float32),
    )(h, pm, bat8, lw["wn1a"], lw["wn1b"], lw["bn1"], lw["wn2"],
      lw["bn2"], p["emb_out"]["W"], p["emb_out"]["b"][None, :])


def _head_body(z_ref, w1_ref, b1_ref, w2_ref, b2_ref, wo_ref, bo_ref, o_ref):
    z = z_ref[...]
    r = _silu(z @ w1_ref[...] + b1_ref[...])
    z2 = r @ w2_ref[...] + b2_ref[...] + z
    o_ref[...] = z2 @ wo_ref[...] + bo_ref[...]


def _tc_head(z, p):
    return pl.pallas_call(
        _head_body,
        out_shape=jax.ShapeDtypeStruct((BGRP, PROP), jnp.float32),
    )(z, p["res1"]["W"], p["res1"]["b"][None, :],
      p["res2"]["W"], p["res2"]["b"][None, :],
      p["out"]["W"], p["out"]["b"][None, :])


def _layer_weights(layer):
    w1 = layer["edge1"]["W"]
    return {
        "w1a": w1[:HID],
        "w1b": w1[HID:2 * HID],
        "b1": layer["edge1"]["b"][None, :],
        "r8": jnp.zeros((8, HID), jnp.float32).at[3].set(w1[2 * HID]),
        "we8": jnp.zeros((8, HID), jnp.float32).at[:4].set(w1[2 * HID + 1:]),
        "w2": layer["edge2"]["W"],
        "b2": layer["edge2"]["b"][None, :],
        "wc1": layer["coord1"]["W"],
        "bc1": layer["coord1"]["b"][None, :],
        "wc2_8": jnp.broadcast_to(layer["coord2"]["W"], (HID, 8)),
        "wn1a": layer["node1"]["W"][:HID],
        "wn1b": layer["node1"]["W"][HID:],
        "bn1": layer["node1"]["b"][None, :],
        "wn2": layer["node2"]["W"],
        "bn2": layer["node2"]["b"][None, :],
    }


def kernel(x, pos, edge_index, edge_attr, batch, x1, frag_levels, adduct_feats, params):
    row = edge_index[0].astype(jnp.int32)
    col = edge_index[1].astype(jnp.int32)
    pad = EP - E
    rowg3 = jnp.concatenate(
        [row, jnp.zeros((pad,), jnp.int32)]).reshape(NW, SCH, 128)
    colni3 = jnp.concatenate(
        [col + N, jnp.full((pad,), N, jnp.int32)]).reshape(NW, SCH, 128)
    rows3 = jnp.concatenate(
        [row, jnp.full((pad,), N, jnp.int32)]).reshape(NW, SCH, 128)
    ea_pad = jnp.zeros((EP, 8), jnp.float32).at[:E, :4].set(edge_attr)
    pospad = jnp.zeros((NP, 8), jnp.float32).at[:N, :3].set(pos)
    zf = jnp.zeros((NP, HID), jnp.float32)
    za = jnp.zeros((NP * 8,), jnp.float32)
    bat8 = jnp.broadcast_to(batch.astype(jnp.int32)[:, None], (N, 8))

    lws = [_layer_weights(l) for l in params["layers"]]
    p0 = {"emb_in": params["emb_in"],
          "w1a": lws[0]["w1a"], "w1b": lws[0]["w1b"], "b1": lws[0]["b1"]}

    h, coord, tab = _tc_init(x, pospad, p0)
    for l in range(4):
        gfa, gfb, aux1f = _sc_gather(
            tab.reshape(2 * N, HID), coord.reshape(NP * 8), rowg3, colni3)
        msg, aux2 = _tc_edge(gfa, gfb, aux1f.reshape(EP, 8), ea_pad, lws[l])
        pm = _sc_scatter_m(msg, rows3, zf)
        pa = _sc_scatter_aux(aux2.reshape(EP * 8), rows3, za)
        if l < 3:
            h, coord, tab = _tc_node(h, coord, pm, pa.reshape(NW, NP, 8),
                                     lws[l], lws[l + 1])
        else:
            pooled = _tc_pool(h, pm, bat8, lws[l], params)

    z = jnp.concatenate([pooled, x1, frag_levels.reshape(BGRP, 32),
                         adduct_feats.reshape(BGRP, 32)], axis=1)
    return _tc_head(z, params)
